# baseline (speedup 1.0000x reference)
import functools

import jax
import jax.numpy as jnp
from jax import lax
from jax.experimental import pallas as pl
from jax.experimental.pallas import tpu as pltpu
from jax.experimental.pallas import tpu_sc as plsc

N_NODES = 10000
NP = 10112
N_EDGES = 320000
D = 128
D_OUT = 64

NC = 2
NS = 16
NW = NC * NS

CHUNK = 128
CPW = 79
E_PAD = NW * CPW * CHUNK
PACK = 16384

STRIPE = NP // NS

DEG_LEN = 10240
DSTRIPE = DEG_LEN // NS

BLK = 1264
_GRID = NP // BLK


def _unpack_chunk(packed_ref, j, sbuf, dbuf, b, want_src):
    for k in range(CHUNK // 16):
        v = packed_ref[j, pl.ds(k * 16, 16)]
        dbuf[b, pl.ds(k * 16, 16)] = jnp.bitwise_and(v, PACK - 1)
        if want_src:
            sbuf[b, pl.ds(k * 16, 16)] = lax.shift_right_logical(v, 14)


def _deg_body(packed_hbm, out_hbm, idx_p, ones_v, zstripe, dbuf, deg_sh):
    c = lax.axis_index("c")
    s = lax.axis_index("s")
    wid = c * NS + s

    def fill_ones(i, _):
        ones_v[pl.ds(i * 16, 16)] = jnp.ones((16,), jnp.float32)
        return 0

    lax.fori_loop(0, CHUNK // 16, fill_ones, 0)

    def fill_z(i, _):
        zstripe[pl.ds(i * 16, 16)] = jnp.zeros((16,), jnp.float32)
        return 0

    lax.fori_loop(0, DSTRIPE // 16, fill_z, 0)

    pltpu.sync_copy(packed_hbm.at[wid], idx_p)
    pltpu.sync_copy(zstripe, deg_sh.at[pl.ds(s * DSTRIPE, DSTRIPE)])
    plsc.subcore_barrier()

    def body(j, _):
        _unpack_chunk(idx_p, j, None, dbuf, 0, want_src=False)
        pltpu.sync_copy(ones_v, deg_sh.at[dbuf.at[0]], add=True)
        return 0

    lax.fori_loop(0, CPW, body, 0)
    plsc.subcore_barrier()
    pltpu.sync_copy(
        deg_sh.at[pl.ds(s * DSTRIPE, DSTRIPE)],
        out_hbm.at[c].at[pl.ds(s * DSTRIPE, DSTRIPE)],
    )


_deg_kernel = functools.partial(
    pl.kernel,
    out_type=jax.ShapeDtypeStruct((NC, DEG_LEN), jnp.float32),
    mesh=plsc.VectorSubcoreMesh(core_axis_name="c", subcore_axis_name="s"),
    scratch_types=[
        pltpu.VMEM((CPW, CHUNK), jnp.int32),
        pltpu.VMEM((CHUNK,), jnp.float32),
        pltpu.VMEM((DSTRIPE,), jnp.float32),
        pltpu.VMEM((1, CHUNK), jnp.int32),
        pltpu.VMEM_SHARED((DEG_LEN,), jnp.float32),
    ],
)(_deg_body)


def _agg_body(hs_hbm, packed_hbm, out_hbm, idx_p, sbuf, dbuf, rows, acc_sh, gsem, ssem):
    c = lax.axis_index("c")
    s = lax.axis_index("s")
    wid = c * NS + s

    def fill_z(i, _):
        rows[0, i // 8, pl.ds((i % 8) * 16, 16)] = jnp.zeros((16,), jnp.float32)
        return 0

    lax.fori_loop(0, CHUNK * 8, fill_z, 0)

    pltpu.sync_copy(packed_hbm.at[wid], idx_p)
    for k in range(4):
        pltpu.sync_copy(rows.at[0], acc_sh.at[pl.ds(s * STRIPE + k * CHUNK, CHUNK)])
    pltpu.sync_copy(
        rows.at[0].at[pl.ds(0, STRIPE - 4 * CHUNK)],
        acc_sh.at[pl.ds(s * STRIPE + 4 * CHUNK, STRIPE - 4 * CHUNK)],
    )
    plsc.subcore_barrier()

    def wait_gather():
        pltpu.make_async_copy(hs_hbm.at[sbuf.at[0]], rows.at[0], gsem).wait()

    def wait_scatter():
        pltpu.make_async_copy(rows.at[0], acc_sh.at[dbuf.at[0]], ssem).wait()

    _unpack_chunk(idx_p, 0, sbuf, dbuf, 0, want_src=True)
    pltpu.async_copy(hs_hbm.at[sbuf.at[0]], rows.at[0], gsem)

    def body(j, _):
        wait_gather()
        @pl.when(j >= 1)
        def _():
            wait_scatter()

        _unpack_chunk(idx_p, j + 1, sbuf, dbuf, (j + 1) % 2, want_src=True)
        pltpu.async_copy(hs_hbm.at[sbuf.at[(j + 1) % 2]], rows.at[(j + 1) % 2], gsem)
        pltpu.async_copy(rows.at[j % 2], acc_sh.at[dbuf.at[j % 2]], ssem, add=True)
        return 0

    lax.fori_loop(0, CPW - 1, body, 0)
    wait_gather()
    wait_scatter()
    pltpu.async_copy(
        rows.at[(CPW - 1) % 2], acc_sh.at[dbuf.at[(CPW - 1) % 2]], ssem, add=True
    )
    wait_scatter()

    plsc.subcore_barrier()
    pltpu.sync_copy(
        acc_sh.at[pl.ds(s * STRIPE, STRIPE)],
        out_hbm.at[c].at[pl.ds(s * STRIPE, STRIPE)],
    )


_agg_kernel = functools.partial(
    pl.kernel,
    out_type=jax.ShapeDtypeStruct((NC, NP, D), jnp.float32),
    mesh=plsc.VectorSubcoreMesh(core_axis_name="c", subcore_axis_name="s"),
    scratch_types=[
        pltpu.VMEM((CPW, CHUNK), jnp.int32),
        pltpu.VMEM((2, CHUNK), jnp.int32),
        pltpu.VMEM((2, CHUNK), jnp.int32),
        pltpu.VMEM((2, CHUNK, D), jnp.float32),
        pltpu.VMEM_SHARED((NP, D), jnp.float32),
        pltpu.SemaphoreType.DMA,
        pltpu.SemaphoreType.DMA,
    ],
)(_agg_body)


def _dinv(deg_ref):
    return lax.rsqrt(jnp.sum(deg_ref[...], axis=1) + 1.0)


def _prep_body(x_ref, w_ref, deg_ref, o_ref):
    dinv = _dinv(deg_ref)
    h = jnp.dot(x_ref[...], w_ref[...], preferred_element_type=jnp.float32)
    o_ref[...] = h * dinv[:, None]


def _mid_body(agg_ref, hs_ref, deg_ref, b_ref, w_ref, o_ref):
    dinv = _dinv(deg_ref)
    p = (agg_ref[0] + agg_ref[1] + hs_ref[...]) * dinv[:, None] + b_ref[...]
    h = jnp.maximum(p, 0.0)
    o_ref[...] = jnp.dot(h, w_ref[...], preferred_element_type=jnp.float32) * dinv[:, None]


def _final_body(agg_ref, hs_ref, deg_ref, b_ref, wc_ref, bc_ref, o_ref, acc):
    i = pl.program_id(0)
    dinv = _dinv(deg_ref)
    p = (agg_ref[0] + agg_ref[1] + hs_ref[...]) * dinv[:, None] + b_ref[...]
    h = jnp.maximum(p, 0.0)
    row = lax.broadcasted_iota(jnp.int32, (BLK, 1), 0) + i * BLK
    h = jnp.where(row < N_NODES, h, 0.0)
    part = jnp.sum(h, axis=0, keepdims=True)

    @pl.when(i == 0)
    def _():
        acc[...] = part

    @pl.when(i > 0)
    def _():
        acc[...] = acc[...] + part

    @pl.when(i == pl.num_programs(0) - 1)
    def _():
        pooled = acc[...] * (1.0 / N_NODES)
        o_ref[...] = (
            jnp.dot(pooled, wc_ref[...], preferred_element_type=jnp.float32)
            + bc_ref[...]
        )


_row_spec = pl.BlockSpec((BLK, D), lambda i: (i, 0))
_w_spec = pl.BlockSpec((D, D), lambda i: (0, 0))
_deg_spec = pl.BlockSpec((BLK, NC), lambda i: (i, 0))
_agg_spec = pl.BlockSpec((NC, BLK, D), lambda i: (0, i, 0))
_b_spec = pl.BlockSpec((D,), lambda i: (0,))

_rows_t = jax.ShapeDtypeStruct((NP, D), jnp.float32)

_prep = pl.pallas_call(
    _prep_body,
    grid=(_GRID,),
    in_specs=[_row_spec, _w_spec, _deg_spec],
    out_specs=_row_spec,
    out_shape=_rows_t,
)

_mid = pl.pallas_call(
    _mid_body,
    grid=(_GRID,),
    in_specs=[_agg_spec, _row_spec, _deg_spec, _b_spec, _w_spec],
    out_specs=_row_spec,
    out_shape=_rows_t,
)

_final = pl.pallas_call(
    _final_body,
    grid=(_GRID,),
    in_specs=[
        _agg_spec,
        _row_spec,
        _deg_spec,
        _b_spec,
        pl.BlockSpec((D, D_OUT), lambda i: (0, 0)),
        pl.BlockSpec((D_OUT,), lambda i: (0,)),
    ],
    out_specs=pl.BlockSpec((1, D_OUT), lambda i: (0, 0)),
    out_shape=jax.ShapeDtypeStruct((1, D_OUT), jnp.float32),
    scratch_shapes=[pltpu.VMEM((1, D), jnp.float32)],
)


def kernel(x, edge_index, W1, b1, W2, b2, Wc, bc):
    src = edge_index[0].astype(jnp.int32)
    dst = edge_index[1].astype(jnp.int32)
    npad = E_PAD - N_EDGES
    pad_dst = N_NODES + jnp.arange(npad, dtype=jnp.int32) % (NP - N_NODES)
    src = jnp.concatenate([src, jnp.zeros((npad,), jnp.int32)])
    dst = jnp.concatenate([dst, pad_dst])
    packed = (src * PACK + dst).reshape(NW, CPW, CHUNK)

    xp = jnp.zeros((NP, D), jnp.float32).at[:N_NODES].set(x)

    degp = _deg_kernel(packed)
    deg = jnp.zeros((NP, NC), jnp.float32).at[:N_NODES].set(degp[:, :N_NODES].T)

    hs1 = _prep(xp, W1, deg)
    agg1 = _agg_kernel(hs1, packed)
    hs2 = _mid(agg1, hs1, deg, b1, W2)
    agg2 = _agg_kernel(hs2, packed)
    out = _final(agg2, hs2, deg, b2, Wc, bc)
    return out.reshape(D_OUT)
